# baseline (speedup 1.0000x reference)
import functools

import jax
import jax.numpy as jnp
from jax import lax
from jax.experimental import pallas as pl
from jax.experimental.pallas import tpu as pltpu
from jax.experimental.pallas import tpu_sc as plsc

NC = 2
NS = 16
LANES = 16
NW = NC * NS


def _worker_id():
  return lax.axis_index("s") * NC + lax.axis_index("c")


def _iota():
  return lax.iota(jnp.int32, LANES)


def _splat(x):
  return jnp.broadcast_to(jnp.asarray(x, jnp.int32), (LANES,))


def _make_prep(BL, chunk):
  nvec = chunk // LANES
  mesh = plsc.VectorSubcoreMesh(
      core_axis_name="c", subcore_axis_name="s", num_cores=NC,
      num_subcores=NS)

  @functools.partial(
      pl.kernel,
      out_type=(
          jax.ShapeDtypeStruct((NW, LANES), jnp.int32),
          jax.ShapeDtypeStruct((BL,), jnp.int32),
          jax.ShapeDtypeStruct((BL,), jnp.int32),
          jax.ShapeDtypeStruct((BL,), jnp.int32),
      ),
      mesh=mesh,
      compiler_params=pltpu.CompilerParams(needs_layout_passes=False),
      scratch_types=(
          pltpu.VMEM((chunk,), jnp.int32),
          pltpu.VMEM((chunk,), jnp.int32),
          pltpu.VMEM((chunk,), jnp.int32),
          pltpu.VMEM((chunk,), jnp.int32),
          pltpu.VMEM((LANES,), jnp.int32),
          pltpu.VMEM((LANES,), jnp.int32),
      ),
  )
  def prep(ids_hbm, img_hbm, counts_hbm, mtok_hbm, gidx_hbm, dest_hbm,
           ids_v, mtok_v, gidx_v, dest_v, img_v, cnt_v):
    w = _worker_id()
    base = w * chunk
    pltpu.sync_copy(ids_hbm.at[pl.ds(base, chunk)], ids_v)
    pltpu.sync_copy(img_hbm, img_v)
    img = img_v[...]
    it = _iota()
    basev = _splat(base)
    carry_m = _splat(0)
    carry_u = _splat(0)
    one = _splat(1)
    zero = _splat(0)
    for i in range(nvec):
      idv = ids_v[pl.ds(i * LANES, LANES)]
      tokv = basev + _splat(i * LANES) + it
      m = idv == img
      um = idv != img
      mi = jnp.where(m, one, zero)
      ui = jnp.where(um, one, zero)
      ranks_m = carry_m + plsc.cumsum(mi) - one
      ranks_u = carry_u + plsc.cumsum(ui) - one
      plsc.store_scatter(mtok_v, [ranks_m], tokv, mask=m)
      plsc.store_scatter(gidx_v, [ranks_u], idv, mask=um)
      plsc.store_scatter(dest_v, [ranks_u], tokv, mask=um)
      carry_m = carry_m + _splat(jnp.sum(mi))
      carry_u = carry_u + _splat(jnp.sum(ui))
    cnt_v[...] = carry_m
    uw = carry_u[0]
    pad_n = (LANES - uw % LANES) % LANES
    last = jnp.minimum(jnp.maximum(uw - 1, 0), chunk - 1)
    dup_g = plsc.load_gather(gidx_v, [_splat(last)])
    dup_d = plsc.load_gather(dest_v, [_splat(last)])
    slots = jnp.minimum(_splat(uw) + it, _splat(chunk - 1))
    padm = it < _splat(pad_n)
    plsc.store_scatter(gidx_v, [slots], dup_g, mask=padm)
    plsc.store_scatter(dest_v, [slots], dup_d, mask=padm)
    pltpu.sync_copy(mtok_v, mtok_hbm.at[pl.ds(base, chunk)])
    pltpu.sync_copy(gidx_v, gidx_hbm.at[pl.ds(base, chunk)])
    pltpu.sync_copy(dest_v, dest_hbm.at[pl.ds(base, chunk)])
    pltpu.sync_copy(cnt_v, counts_hbm.at[w])

  return prep


def _make_main(BL, chunk, V, D, Nv):
  nwin = chunk // LANES
  NBUF = 7
  mesh = plsc.VectorSubcoreMesh(
      core_axis_name="c", subcore_axis_name="s", num_cores=NC,
      num_subcores=NS)

  @functools.partial(
      pl.kernel,
      out_type=jax.ShapeDtypeStruct((BL, D), jnp.float32),
      mesh=mesh,
      compiler_params=pltpu.CompilerParams(needs_layout_passes=False),
      scratch_types=(
          pltpu.VMEM((NBUF, LANES, D), jnp.float32),
          pltpu.VMEM((chunk,), jnp.int32),
          pltpu.VMEM((chunk,), jnp.int32),
          pltpu.VMEM((NW, LANES), jnp.int32),
          pltpu.VMEM((LANES,), jnp.int32),
          pltpu.SemaphoreType.DMA,
          pltpu.SemaphoreType.DMA,
          pltpu.SemaphoreType.DMA,
          pltpu.SemaphoreType.DMA,
          pltpu.SemaphoreType.DMA,
          pltpu.SemaphoreType.DMA,
          pltpu.SemaphoreType.DMA,
          pltpu.SemaphoreType.DMA,
          pltpu.SemaphoreType.DMA,
          pltpu.SemaphoreType.DMA,
          pltpu.SemaphoreType.DMA,
          pltpu.SemaphoreType.DMA,
          pltpu.SemaphoreType.DMA,
          pltpu.SemaphoreType.DMA,
          pltpu.SemaphoreType.DMA,
          pltpu.SemaphoreType.DMA,
          pltpu.SemaphoreType.DMA,
      ),
  )
  def main(w_hbm, vis_hbm, gidx_hbm, dest_hbm, counts_hbm, mtok_hbm,
           out_hbm, rows_v, gidx_v, dest_v, cnt_v, tok_v,
           g0, g1, g2, g3, g4, g5, g6, s0, s1, s2, s3, s4, s5, s6,
           v0, v1, v2):
    w = _worker_id()
    base = w * chunk
    it = _iota()
    gsem = [g0, g1, g2, g3, g4, g5, g6]
    ssem = [s0, s1, s2, s3, s4, s5, s6]
    vis_v = rows_v.at[NBUF - 1]
    cp_g = pltpu.make_async_copy(gidx_hbm.at[pl.ds(base, chunk)], gidx_v, v0)
    cp_d = pltpu.make_async_copy(dest_hbm.at[pl.ds(base, chunk)], dest_v, v1)
    cp_c = pltpu.make_async_copy(counts_hbm, cnt_v, v2)
    cp_g.start()
    cp_d.start()
    cp_c.start()
    cp_c.wait()
    cp_g.wait()
    cp_d.wait()
    cnt_w = cnt_v[w, :][0]
    uw = chunk - cnt_w

    def gather_win(g, b):
      @pl.when(g * LANES < uw)
      def _():
        idx16 = gidx_v[pl.ds(g * LANES, LANES)]
        pltpu.make_async_copy(w_hbm.at[idx16], rows_v.at[b], gsem[b]).start()

    def wait_gather(g, b):
      @pl.when(g * LANES < uw)
      def _():
        pltpu.make_async_copy(w_hbm.at[gidx_v[pl.ds(g * LANES, LANES)]],
                              rows_v.at[b], gsem[b]).wait()

    def scatter_win(g, b):
      @pl.when(g * LANES < uw)
      def _():
        dest16 = dest_v[pl.ds(g * LANES, LANES)]
        pltpu.make_async_copy(rows_v.at[b], out_hbm.at[dest16],
                              ssem[b]).start()

    def wait_scatter(g, b):
      @pl.when(g * LANES < uw)
      def _():
        dest16 = dest_v[pl.ds(g * LANES, LANES)]
        pltpu.make_async_copy(rows_v.at[b], out_hbm.at[dest16],
                              ssem[b]).wait()

    for g in range(min(NBUF - 1, nwin)):
      gather_win(g, g % NBUF)

    prefix = [jnp.int32(0)]
    for j in range(NW):
      prefix.append(prefix[-1] + cnt_v[j, :][0])
    total = prefix[NW]
    ngroups = (total + LANES - 1) // LANES
    niter = jnp.maximum(0, (ngroups - w + NW - 1) // NW)

    def body(i, _):
      g = w + NW * i
      r = _splat(g * LANES) + it
      rr = jnp.minimum(r, _splat(total - 1))
      jv = _splat(0)
      for j in range(1, NW):
        jv = jv + jnp.where(rr >= _splat(prefix[j]), _splat(1), _splat(0))
      adj = _splat(0)
      for j in range(NW):
        adj = jnp.where(jv == _splat(j), _splat(j * chunk - prefix[j]), adj)
      midx = jnp.minimum(jnp.maximum(rr + adj, _splat(0)), _splat(BL - 1))
      visidx = jnp.minimum(rr, _splat(Nv - 1))
      h_tok = pltpu.async_copy(mtok_hbm.at[midx], tok_v, v0)
      h_vis = pltpu.async_copy(vis_hbm.at[visidx], vis_v, v1)
      h_tok.wait()
      h_vis.wait()
      dest16 = tok_v[...]
      pltpu.async_copy(vis_v, out_hbm.at[dest16], v2).wait()
      return 0

    lax.fori_loop(0, niter, body, 0)

    for g in range(nwin):
      b = g % NBUF
      wait_gather(g, b)
      scatter_win(g, b)
      ng = g + NBUF - 1
      if ng < nwin:
        nb = ng % NBUF
        if ng - NBUF >= 0:
          wait_scatter(ng - NBUF, nb)
        gather_win(ng, nb)
    for g in range(max(0, nwin - NBUF), nwin):
      wait_scatter(g, g % NBUF)

  return main


def kernel(input_ids, visual_embeds, W, image_token_id):
  B, L = input_ids.shape
  BL = B * L
  V, D = W.shape
  Nv = visual_embeds.shape[0]
  chunk = BL // NW

  ids = input_ids.reshape(BL).astype(jnp.int32)
  img_vec = jnp.full((LANES,), image_token_id, dtype=jnp.int32)

  counts, mtok, gidx, dest = _make_prep(BL, chunk)(ids, img_vec)
  out = _make_main(BL, chunk, V, D, Nv)(
      W, visual_embeds, gidx, dest, counts, mtok)
  return out.reshape(B, L, D)


# speedup vs baseline: 1.0342x; 1.0342x over previous
import functools

import jax
import jax.numpy as jnp
from jax import lax
from jax.experimental import pallas as pl
from jax.experimental.pallas import tpu as pltpu
from jax.experimental.pallas import tpu_sc as plsc

NC = 2
NS = 16
LANES = 16
NW = NC * NS
WIN = 32


def _worker_id():
  return lax.axis_index("s") * NC + lax.axis_index("c")


def _iota():
  return lax.iota(jnp.int32, LANES)


def _splat(x):
  return jnp.broadcast_to(jnp.asarray(x, jnp.int32), (LANES,))


def _make_prep(BL, chunk):
  nvec = chunk // LANES
  mesh = plsc.VectorSubcoreMesh(
      core_axis_name="c", subcore_axis_name="s", num_cores=NC,
      num_subcores=NS)

  @functools.partial(
      pl.kernel,
      out_type=(
          jax.ShapeDtypeStruct((NW, LANES), jnp.int32),
          jax.ShapeDtypeStruct((BL,), jnp.int32),
          jax.ShapeDtypeStruct((BL,), jnp.int32),
          jax.ShapeDtypeStruct((BL,), jnp.int32),
      ),
      mesh=mesh,
      compiler_params=pltpu.CompilerParams(needs_layout_passes=False),
      scratch_types=(
          pltpu.VMEM((chunk,), jnp.int32),
          pltpu.VMEM((chunk,), jnp.int32),
          pltpu.VMEM((chunk,), jnp.int32),
          pltpu.VMEM((chunk,), jnp.int32),
          pltpu.VMEM((LANES,), jnp.int32),
          pltpu.VMEM((LANES,), jnp.int32),
      ),
  )
  def prep(ids_hbm, img_hbm, counts_hbm, mtok_hbm, gidx_hbm, dest_hbm,
           ids_v, mtok_v, gidx_v, dest_v, img_v, cnt_v):
    w = _worker_id()
    base = w * chunk
    pltpu.sync_copy(ids_hbm.at[pl.ds(base, chunk)], ids_v)
    pltpu.sync_copy(img_hbm, img_v)
    img = img_v[...]
    it = _iota()
    basev = _splat(base)
    carry_m = _splat(0)
    carry_u = _splat(0)
    one = _splat(1)
    zero = _splat(0)
    for i in range(nvec):
      idv = ids_v[pl.ds(i * LANES, LANES)]
      tokv = basev + _splat(i * LANES) + it
      m = idv == img
      um = idv != img
      mi = jnp.where(m, one, zero)
      ui = jnp.where(um, one, zero)
      ranks_m = carry_m + plsc.cumsum(mi) - one
      ranks_u = carry_u + plsc.cumsum(ui) - one
      plsc.store_scatter(mtok_v, [ranks_m], tokv, mask=m)
      plsc.store_scatter(gidx_v, [ranks_u], idv, mask=um)
      plsc.store_scatter(dest_v, [ranks_u], tokv, mask=um)
      carry_m = carry_m + _splat(jnp.sum(mi))
      carry_u = carry_u + _splat(jnp.sum(ui))
    cnt_v[...] = carry_m
    uw = carry_u[0]
    pad_n = (WIN - uw % WIN) % WIN
    last = jnp.minimum(jnp.maximum(uw - 1, 0), chunk - 1)
    dup_g = plsc.load_gather(gidx_v, [_splat(last)])
    dup_d = plsc.load_gather(dest_v, [_splat(last)])
    for h in range(WIN // LANES):
      slots = jnp.minimum(_splat(uw + h * LANES) + it, _splat(chunk - 1))
      padm = (it + _splat(h * LANES)) < _splat(pad_n)
      plsc.store_scatter(gidx_v, [slots], dup_g, mask=padm)
      plsc.store_scatter(dest_v, [slots], dup_d, mask=padm)
    pltpu.sync_copy(mtok_v, mtok_hbm.at[pl.ds(base, chunk)])
    pltpu.sync_copy(gidx_v, gidx_hbm.at[pl.ds(base, chunk)])
    pltpu.sync_copy(dest_v, dest_hbm.at[pl.ds(base, chunk)])
    pltpu.sync_copy(cnt_v, counts_hbm.at[w])

  return prep


def _make_main(BL, chunk, V, D, Nv):
  nwin = chunk // WIN
  NBUF = 3
  mesh = plsc.VectorSubcoreMesh(
      core_axis_name="c", subcore_axis_name="s", num_cores=NC,
      num_subcores=NS)

  @functools.partial(
      pl.kernel,
      out_type=jax.ShapeDtypeStruct((BL, D), jnp.float32),
      mesh=mesh,
      compiler_params=pltpu.CompilerParams(needs_layout_passes=False),
      scratch_types=(
          pltpu.VMEM((NBUF, WIN, D), jnp.float32),
          pltpu.VMEM((LANES, D), jnp.float32),
          pltpu.VMEM((nwin, WIN), jnp.int32),
          pltpu.VMEM((nwin, WIN), jnp.int32),
          pltpu.VMEM((NW, LANES), jnp.int32),
          pltpu.VMEM((LANES,), jnp.int32),
          pltpu.SemaphoreType.DMA,
          pltpu.SemaphoreType.DMA,
          pltpu.SemaphoreType.DMA,
          pltpu.SemaphoreType.DMA,
          pltpu.SemaphoreType.DMA,
          pltpu.SemaphoreType.DMA,
          pltpu.SemaphoreType.DMA,
          pltpu.SemaphoreType.DMA,
          pltpu.SemaphoreType.DMA,
      ),
  )
  def main(w_hbm, vis_hbm, gidx_hbm, dest_hbm, counts_hbm, mtok_hbm,
           out_hbm, rows_v, vis_v, gidx_v, dest_v, cnt_v, tok_v,
           g0, g1, g2, s0, s1, s2, v0, v1, v2):
    w = _worker_id()
    it = _iota()
    gsem = [g0, g1, g2]
    ssem = [s0, s1, s2]
    cp_g = pltpu.make_async_copy(gidx_hbm.at[w], gidx_v, v0)
    cp_d = pltpu.make_async_copy(dest_hbm.at[w], dest_v, v1)
    cp_c = pltpu.make_async_copy(counts_hbm, cnt_v, v2)
    cp_g.start()
    cp_d.start()
    cp_c.start()
    cp_c.wait()
    cp_g.wait()
    cp_d.wait()
    cnt_w = cnt_v[w, :][0]
    uw = chunk - cnt_w

    def gather_win(g, b):
      @pl.when(g * WIN < uw)
      def _():
        pltpu.make_async_copy(w_hbm.at[gidx_v.at[g]], rows_v.at[b],
                              gsem[b]).start()

    def wait_gather(g, b):
      @pl.when(g * WIN < uw)
      def _():
        pltpu.make_async_copy(w_hbm.at[gidx_v.at[g]], rows_v.at[b],
                              gsem[b]).wait()

    def scatter_win(g, b):
      @pl.when(g * WIN < uw)
      def _():
        pltpu.make_async_copy(rows_v.at[b], out_hbm.at[dest_v.at[g]],
                              ssem[b]).start()

    def wait_scatter(g, b):
      @pl.when(g * WIN < uw)
      def _():
        pltpu.make_async_copy(rows_v.at[b], out_hbm.at[dest_v.at[g]],
                              ssem[b]).wait()

    for g in range(min(NBUF - 1, nwin)):
      gather_win(g, g % NBUF)

    prefix = [jnp.int32(0)]
    for j in range(NW):
      prefix.append(prefix[-1] + cnt_v[j, :][0])
    total = prefix[NW]
    ngroups = (total + LANES - 1) // LANES
    niter = jnp.maximum(0, (ngroups - w + NW - 1) // NW)

    def body(i, _):
      g = w + NW * i
      r = _splat(g * LANES) + it
      rr = jnp.minimum(r, _splat(total - 1))
      jv = _splat(0)
      for j in range(1, NW):
        jv = jv + jnp.where(rr >= _splat(prefix[j]), _splat(1), _splat(0))
      adj = _splat(0)
      for j in range(NW):
        adj = jnp.where(jv == _splat(j), _splat(j * chunk - prefix[j]), adj)
      midx = jnp.minimum(jnp.maximum(rr + adj, _splat(0)), _splat(BL - 1))
      visidx = jnp.minimum(rr, _splat(Nv - 1))
      h_tok = pltpu.async_copy(mtok_hbm.at[midx], tok_v, v0)
      h_vis = pltpu.async_copy(vis_hbm.at[visidx], vis_v, v1)
      h_tok.wait()
      h_vis.wait()
      dest16 = tok_v[...]
      pltpu.async_copy(vis_v, out_hbm.at[dest16], v2).wait()
      return 0

    lax.fori_loop(0, niter, body, 0)

    for g in range(nwin):
      b = g % NBUF
      wait_gather(g, b)
      scatter_win(g, b)
      ng = g + NBUF - 1
      if ng < nwin:
        nb = ng % NBUF
        if ng - NBUF >= 0:
          wait_scatter(ng - NBUF, nb)
        gather_win(ng, nb)
    for g in range(max(0, nwin - NBUF), nwin):
      wait_scatter(g, g % NBUF)

  return main


def kernel(input_ids, visual_embeds, W, image_token_id):
  B, L = input_ids.shape
  BL = B * L
  V, D = W.shape
  Nv = visual_embeds.shape[0]
  chunk = BL // NW

  ids = input_ids.reshape(BL).astype(jnp.int32)
  img_vec = jnp.full((LANES,), image_token_id, dtype=jnp.int32)

  counts, mtok, gidx, dest = _make_prep(BL, chunk)(ids, img_vec)
  nwin = chunk // WIN
  gidx = gidx.reshape(NW, nwin, WIN)
  dest = dest.reshape(NW, nwin, WIN)
  out = _make_main(BL, chunk, V, D, Nv)(
      W, visual_embeds, gidx, dest, counts, mtok)
  return out.reshape(B, L, D)
